# baseline (speedup 1.0000x reference)
import dataclasses
import functools

import jax
import jax.numpy as jnp
from jax import lax
from jax.experimental import pallas as pl
from jax.experimental.pallas import tpu as pltpu
from jax.experimental.pallas import tpu_sc as plsc

N = 10000
E = 160000
E_PAD = 163840
EB = 64
NB_ALL = E_PAD // EB
NB_T = NB_ALL // 16
NSLOT = 8
CW = 64
BM = 400
ROWS_T = 624
TAIL0 = 16 * ROWS_T

_DIMS = [(128, 500), (500, 1000), (1000, 1000), (1000, 1000), (1000, 1000),
         (1000, 1000), (1000, 128)]


def _rup(v, m=128):
    return (v + m - 1) // m * m


def _make_prop(n_chunks, j0=0, n_rounds=None):
    assert n_chunks % 2 == 0
    if n_rounds is None:
        n_rounds = n_chunks // 2
    n_out = 2 * n_rounds

    mesh = plsc.VectorSubcoreMesh(core_axis_name="c", subcore_axis_name="s")
    cp = pltpu.CompilerParams()
    if "needs_layout_passes" in pltpu.CompilerParams.__dataclass_fields__:
        cp = dataclasses.replace(cp, needs_layout_passes=False)
    if "use_tc_tiling_on_sc" in pltpu.CompilerParams.__dataclass_fields__:
        cp = dataclasses.replace(cp, use_tc_tiling_on_sc=False)

    @functools.partial(
        pl.kernel,
        compiler_params=cp,
        out_type=jax.ShapeDtypeStruct((n_out, N, CW), jnp.float32),
        mesh=mesh,
        scratch_types=(
            [
                pltpu.VMEM((NB_T, EB), jnp.int32),
                pltpu.VMEM((NB_T, EB), jnp.int32),
                pltpu.VMEM((NB_T, EB), jnp.float32),
                pltpu.VMEM((128, CW), jnp.float32),
                pltpu.VMEM_SHARED((N, CW), jnp.float32),
                pltpu.VMEM((NSLOT * EB, CW), jnp.float32),
                pltpu.SemaphoreType.DMA((NSLOT,)),
                pltpu.SemaphoreType.DMA((NSLOT,)),
            ]
        ),
    )
    def prop(g_hbm, src_hbm, dst_hbm, ew_hbm, out_hbm,
             sidx, didx, ewb, zbuf, acc, ring, sg, ss):
        cid = lax.axis_index("c")
        tid = lax.axis_index("s")

        zero = jnp.zeros((16,), jnp.float32)

        @plsc.parallel_loop(0, 128, unroll=4)
        def _(r):
            for k2 in range(CW // 16):
                zbuf[r, pl.ds(16 * k2, 16)] = zero

        row0 = pl.multiple_of(tid * NB_T, 8)
        pltpu.sync_copy(src_hbm.at[pl.ds(row0, NB_T)], sidx)
        pltpu.sync_copy(dst_hbm.at[pl.ds(row0, NB_T)], didx)
        pltpu.sync_copy(ew_hbm.at[pl.ds(row0, NB_T)], ewb)

        def scale(k, i):
            base_r = k * EB

            @plsc.parallel_loop(0, EB, unroll=8)
            def _(r):
                spl = plsc.load_gather(
                    ewb, [lax.broadcast(i, (16,)), lax.broadcast(r, (16,))])
                rr = base_r + r
                for k2 in range(CW // 16):
                    sl = pl.ds(16 * k2, 16)
                    ring[rr, sl] = ring[rr, sl] * spl

        @pl.loop(0, n_rounds)
        def _(j):
            fc = 2 * (j0 + j) + cid
            oc_i = 2 * j + cid
            gtab = g_hbm.at[fc]

            def slot(k):
                return ring.at[pl.ds(pl.multiple_of(k * EB, 8), EB)]

            def g_start(i, k):
                pltpu.async_copy(gtab.at[sidx.at[i]], slot(k), sg.at[k])

            def g_wait(i, k):
                pltpu.make_async_copy(gtab.at[sidx.at[i]], slot(k),
                                      sg.at[k]).wait()

            def s_start(i, k):
                pltpu.async_copy(slot(k), acc.at[didx.at[i]], ss.at[k],
                                 add=True)

            def s_wait(i, k):
                pltpu.make_async_copy(slot(k), acc.at[didx.at[i]],
                                      ss.at[k]).wait()

            base = pl.multiple_of(tid * ROWS_T, 8)
            for off, sz in ((0, 128), (128, 128), (256, 128), (384, 128),
                            (512, 112)):
                pltpu.sync_copy(zbuf.at[pl.ds(0, sz)],
                                acc.at[pl.ds(base + off, sz)])

            @pl.when(tid < 2)
            def _():
                toff = pl.multiple_of(TAIL0 + tid * 8, 8)
                pltpu.sync_copy(zbuf.at[pl.ds(0, 8)], acc.at[pl.ds(toff, 8)])

            plsc.subcore_barrier()

            @pl.loop(0, 4)
            def _(z):
                g_start(z, z)

            @pl.loop(0, NB_T)
            def _(i):
                k = lax.rem(i, 8)
                g_wait(i, k)
                scale(k, i)
                s_start(i, k)
                jn = i + 4
                kn = lax.rem(jn, 8)

                @pl.when(jn < NB_T)
                def _():
                    @pl.when(jn >= 8)
                    def _():
                        s_wait(jn - 8, kn)

                    g_start(jn, kn)

            @pl.loop(0, 8)
            def _(k):
                s_wait(NB_T - 8 + k, k)

            plsc.subcore_barrier()
            out_c = out_hbm.at[oc_i]
            pltpu.sync_copy(acc.at[pl.ds(base, ROWS_T)],
                            out_c.at[pl.ds(base, ROWS_T)])

            @pl.when(tid < 2)
            def _():
                toff = pl.multiple_of(TAIL0 + tid * 8, 8)
                pltpu.sync_copy(acc.at[pl.ds(toff, 8)],
                                out_c.at[pl.ds(toff, 8)])

    return prop


_DOT = functools.partial(lax.dot_general,
                         dimension_numbers=(((1,), (0,)), ((), ())),
                         precision=lax.Precision.HIGHEST,
                         preferred_element_type=jnp.float32)


def _k0(x, sdeg):
    def body(s_ref, x_ref, dv_ref, g0_ref):
        deg = s_ref[0] + 1.0
        dv = jnp.where(deg > 0, lax.rsqrt(deg), 0.0)
        dv_ref[...] = dv
        for c in range(2):
            g0_ref[c] = dv * x_ref[:, CW * c:CW * (c + 1)]

    return pl.pallas_call(
        body,
        grid=(N // BM,),
        in_specs=[
            pl.BlockSpec((1, BM, CW), lambda i: (0, i, 0)),
            pl.BlockSpec((BM, 128), lambda i: (i, 0)),
        ],
        out_specs=[
            pl.BlockSpec((BM, CW), lambda i: (i, 0)),
            pl.BlockSpec((2, BM, CW), lambda i: (0, i, 0)),
        ],
        out_shape=[
            jax.ShapeDtypeStruct((N, CW), jnp.float32),
            jax.ShapeDtypeStruct((2, N, CW), jnp.float32),
        ],
    )(sdeg, x)


def _make_mid(cin, cout, relu, scale_out):
    npad = cout * CW

    def body(s_ref, g_ref, dv_ref, w_ref, b_ref, o_ref):
        dv = dv_ref[...]
        t = jnp.concatenate(
            [dv * (s_ref[k] + g_ref[k]) for k in range(cin)], axis=1)
        u = _DOT(t, w_ref[...]) + b_ref[0]
        if relu:
            u = jnp.maximum(u, 0.0)
        for c in range(cout):
            oc = u[:, CW * c:CW * (c + 1)]
            o_ref[c] = dv * oc if scale_out else oc

    def run(s, g, dv, w2, b):
        return pl.pallas_call(
            body,
            grid=(N // BM,),
            in_specs=[
                pl.BlockSpec((cin, BM, CW), lambda i: (0, i, 0)),
                pl.BlockSpec((cin, BM, CW), lambda i: (0, i, 0)),
                pl.BlockSpec((BM, CW), lambda i: (i, 0)),
                pl.BlockSpec((cin * CW, npad), lambda i: (0, 0)),
                pl.BlockSpec((1, npad), lambda i: (0, 0)),
            ],
            out_specs=pl.BlockSpec((cout, BM, CW), lambda i: (0, i, 0)),
            out_shape=jax.ShapeDtypeStruct((cout, N, CW), jnp.float32),
        )(s, g, dv, w2, b)

    return run


def _make_mid_a(ch):
    def run(s, g, dv, w2):
        npad = w2.shape[1]

        def body(s_ref, g_ref, dv_ref, w_ref, o_ref):
            dvv = dv_ref[...]
            t = jnp.concatenate(
                [dvv * (s_ref[k] + g_ref[k]) for k in range(ch)], axis=1)
            o_ref[...] = _DOT(t, w_ref[...])

        return pl.pallas_call(
            body,
            grid=(N // BM,),
            in_specs=[
                pl.BlockSpec((ch, BM, CW), lambda i: (0, i, 0)),
                pl.BlockSpec((ch, BM, CW), lambda i: (0, i, 0)),
                pl.BlockSpec((BM, CW), lambda i: (i, 0)),
                pl.BlockSpec((ch * CW, npad), lambda i: (0, 0)),
            ],
            out_specs=pl.BlockSpec((BM, npad), lambda i: (i, 0)),
            out_shape=jax.ShapeDtypeStruct((N, npad), jnp.float32),
        )(s, g, dv, w2)

    return run


def _make_mid_b(ch, cout, relu, scale_out):
    npad = cout * CW

    def body(p_ref, s_ref, g_ref, dv_ref, w_ref, b_ref, o_ref):
        dvv = dv_ref[...]
        t = jnp.concatenate(
            [dvv * (s_ref[k] + g_ref[k]) for k in range(ch)], axis=1)
        u = p_ref[...] + _DOT(t, w_ref[...]) + b_ref[0]
        if relu:
            u = jnp.maximum(u, 0.0)
        for c in range(cout):
            oc = u[:, CW * c:CW * (c + 1)]
            o_ref[c] = dvv * oc if scale_out else oc

    def run(part, s_hi, g, dv, w2, b):
        return pl.pallas_call(
            body,
            grid=(N // BM,),
            in_specs=[
                pl.BlockSpec((BM, npad), lambda i: (i, 0)),
                pl.BlockSpec((ch, BM, CW), lambda i: (0, i, 0)),
                pl.BlockSpec((ch, BM, CW), lambda i: (1, i, 0)),
                pl.BlockSpec((BM, CW), lambda i: (i, 0)),
                pl.BlockSpec((ch * CW, npad), lambda i: (0, 0)),
                pl.BlockSpec((1, npad), lambda i: (0, 0)),
            ],
            out_specs=pl.BlockSpec((cout, BM, CW), lambda i: (0, i, 0)),
            out_shape=jax.ShapeDtypeStruct((cout, N, CW), jnp.float32),
        )(part, s_hi, g, dv, w2, b)

    return run


def _k_last_mm(h6, dv, w2):
    def body(h_ref, dv_ref, w_ref, o_ref):
        t = jnp.concatenate([h_ref[k] for k in range(16)], axis=1)
        acc = _DOT(t, w_ref[...])
        dv = dv_ref[...]
        for c in range(2):
            o_ref[c] = dv * acc[:, CW * c:CW * (c + 1)]

    return pl.pallas_call(
        body,
        grid=(N // BM,),
        in_specs=[
            pl.BlockSpec((16, BM, CW), lambda i: (0, i, 0)),
            pl.BlockSpec((BM, CW), lambda i: (i, 0)),
            pl.BlockSpec((16 * CW, 128), lambda i: (0, 0)),
        ],
        out_specs=pl.BlockSpec((2, BM, CW), lambda i: (0, i, 0)),
        out_shape=jax.ShapeDtypeStruct((2, N, CW), jnp.float32),
    )(h6, dv, w2)


def _k_final(s6, mscaled, dv, b):
    def body(s_ref, m_ref, dv_ref, b_ref, o_ref):
        dv = dv_ref[...]
        for c in range(2):
            o_ref[:, CW * c:CW * (c + 1)] = (
                dv * (s_ref[c] + m_ref[c]) + b_ref[0, CW * c:CW * (c + 1)])

    return pl.pallas_call(
        body,
        grid=(N // BM,),
        in_specs=[
            pl.BlockSpec((2, BM, CW), lambda i: (0, i, 0)),
            pl.BlockSpec((2, BM, CW), lambda i: (0, i, 0)),
            pl.BlockSpec((BM, CW), lambda i: (i, 0)),
            pl.BlockSpec((1, 128), lambda i: (0, 0)),
        ],
        out_specs=pl.BlockSpec((BM, 128), lambda i: (i, 0)),
        out_shape=jax.ShapeDtypeStruct((N, 128), jnp.float32),
    )(s6, mscaled, dv, b)


def kernel(x, edge_index, edge_weight, Ws, bs):
    f32 = jnp.float32
    src = edge_index[0].astype(jnp.int32)
    dst = edge_index[1].astype(jnp.int32)
    ew = edge_weight.astype(f32)
    pad = E_PAD - E
    src_m = jnp.concatenate([src, jnp.zeros((pad,), jnp.int32)]).reshape(NB_ALL, EB)
    dst_m = jnp.concatenate([dst, jnp.zeros((pad,), jnp.int32)]).reshape(NB_ALL, EB)
    ew_m = jnp.concatenate([ew, jnp.zeros((pad,), f32)]).reshape(NB_ALL, EB)

    w2s, b2s = [], []
    for i, (din, dout) in enumerate(_DIMS):
        kp, np_ = _rup(din), _rup(dout)
        w2s.append(jnp.zeros((kp, np_), f32).at[:din, :dout].set(Ws[i]))
        b2s.append(jnp.zeros((1, np_), f32).at[0, :dout].set(bs[i]))

    prop2 = _make_prop(2)
    prop8a = _make_prop(8, 0, 2)
    prop8b = _make_prop(8, 2, 2)
    prop16a = _make_prop(16, 0, 4)
    prop16b = _make_prop(16, 4, 4)

    ones_g = jnp.ones((2, N, CW), f32)
    sdeg = prop2(ones_g, src_m, dst_m, ew_m)
    dv, g = _k0(x, sdeg)

    mid0 = _make_mid(2, 8, True, True)
    midA8 = _make_mid_a(4)
    midB8 = _make_mid_b(4, 16, True, True)
    midA16 = _make_mid_a(8)
    midB16 = _make_mid_b(8, 16, True, True)
    midB16h = _make_mid_b(8, 16, True, False)

    s = prop2(g, src_m, dst_m, ew_m)
    g = mid0(s, g, dv, w2s[0], b2s[0])

    s_lo = prop8a(g, src_m, dst_m, ew_m)
    s_hi = prop8b(g, src_m, dst_m, ew_m)
    part = midA8(s_lo, g, dv, w2s[1][:4 * CW])
    g = midB8(part, s_hi, g, dv, w2s[1][4 * CW:], b2s[1])
    for i in (2, 3, 4):
        s_lo = prop16a(g, src_m, dst_m, ew_m)
        s_hi = prop16b(g, src_m, dst_m, ew_m)
        part = midA16(s_lo, g, dv, w2s[i][:8 * CW])
        g = midB16(part, s_hi, g, dv, w2s[i][8 * CW:], b2s[i])
    s_lo = prop16a(g, src_m, dst_m, ew_m)
    s_hi = prop16b(g, src_m, dst_m, ew_m)
    part = midA16(s_lo, g, dv, w2s[5][:8 * CW])
    h6 = midB16h(part, s_hi, g, dv, w2s[5][8 * CW:], b2s[5])
    mscaled = _k_last_mm(h6, dv, w2s[6])
    s6 = prop2(mscaled, src_m, dst_m, ew_m)
    return _k_final(s6, mscaled, dv, b2s[6])


# speedup vs baseline: 1.0396x; 1.0396x over previous
import dataclasses
import functools

import jax
import jax.numpy as jnp
from jax import lax
from jax.experimental import pallas as pl
from jax.experimental.pallas import tpu as pltpu
from jax.experimental.pallas import tpu_sc as plsc

N = 10000
E = 160000
E_PAD = 163840
EB = 64
NB_ALL = E_PAD // EB
NB_T = NB_ALL // 16
NSLOT = 8
CW = 64
BM = 400
ROWS_T = 624
TAIL0 = 16 * ROWS_T

_DIMS = [(128, 500), (500, 1000), (1000, 1000), (1000, 1000), (1000, 1000),
         (1000, 1000), (1000, 128)]


def _rup(v, m=128):
    return (v + m - 1) // m * m


def _make_prop(n_chunks):
    assert n_chunks % 2 == 0
    n_rounds = n_chunks // 2

    mesh = plsc.VectorSubcoreMesh(core_axis_name="c", subcore_axis_name="s")
    cp = pltpu.CompilerParams()
    if "needs_layout_passes" in pltpu.CompilerParams.__dataclass_fields__:
        cp = dataclasses.replace(cp, needs_layout_passes=False)
    if "use_tc_tiling_on_sc" in pltpu.CompilerParams.__dataclass_fields__:
        cp = dataclasses.replace(cp, use_tc_tiling_on_sc=False)

    @functools.partial(
        pl.kernel,
        compiler_params=cp,
        out_type=jax.ShapeDtypeStruct((n_chunks, N, CW), jnp.float32),
        mesh=mesh,
        scratch_types=(
            [
                pltpu.VMEM((NB_T, EB), jnp.int32),
                pltpu.VMEM((NB_T, EB), jnp.int32),
                pltpu.VMEM((NB_T, EB), jnp.float32),
                pltpu.VMEM((128, CW), jnp.float32),
                pltpu.VMEM_SHARED((N, CW), jnp.float32),
                pltpu.VMEM((NSLOT * EB, CW), jnp.float32),
                pltpu.SemaphoreType.DMA((NSLOT,)),
                pltpu.SemaphoreType.DMA((NSLOT,)),
            ]
        ),
    )
    def prop(g_hbm, src_hbm, dst_hbm, ew_hbm, out_hbm,
             sidx, didx, ewb, zbuf, acc, ring, sg, ss):
        cid = lax.axis_index("c")
        tid = lax.axis_index("s")

        zero = jnp.zeros((16,), jnp.float32)

        @plsc.parallel_loop(0, 128, unroll=4)
        def _(r):
            for k2 in range(CW // 16):
                zbuf[r, pl.ds(16 * k2, 16)] = zero

        row0 = pl.multiple_of(tid * NB_T, 8)
        pltpu.sync_copy(src_hbm.at[pl.ds(row0, NB_T)], sidx)
        pltpu.sync_copy(dst_hbm.at[pl.ds(row0, NB_T)], didx)
        pltpu.sync_copy(ew_hbm.at[pl.ds(row0, NB_T)], ewb)

        def scale(k, i):
            base_r = k * EB

            @plsc.parallel_loop(0, EB, unroll=8)
            def _(r):
                spl = plsc.load_gather(
                    ewb, [lax.broadcast(i, (16,)), lax.broadcast(r, (16,))])
                rr = base_r + r
                for k2 in range(CW // 16):
                    sl = pl.ds(16 * k2, 16)
                    ring[rr, sl] = ring[rr, sl] * spl

        @pl.loop(0, n_rounds)
        def _(j):
            fc = 2 * j + cid
            gtab = g_hbm.at[fc]

            def slot(k):
                return ring.at[pl.ds(pl.multiple_of(k * EB, 8), EB)]

            def g_start(i, k):
                pltpu.async_copy(gtab.at[sidx.at[i]], slot(k), sg.at[k])

            def g_wait(i, k):
                pltpu.make_async_copy(gtab.at[sidx.at[i]], slot(k),
                                      sg.at[k]).wait()

            def s_start(i, k):
                pltpu.async_copy(slot(k), acc.at[didx.at[i]], ss.at[k],
                                 add=True)

            def s_wait(i, k):
                pltpu.make_async_copy(slot(k), acc.at[didx.at[i]],
                                      ss.at[k]).wait()

            base = pl.multiple_of(tid * ROWS_T, 8)
            for off, sz in ((0, 128), (128, 128), (256, 128), (384, 128),
                            (512, 112)):
                pltpu.sync_copy(zbuf.at[pl.ds(0, sz)],
                                acc.at[pl.ds(base + off, sz)])

            @pl.when(tid < 2)
            def _():
                toff = pl.multiple_of(TAIL0 + tid * 8, 8)
                pltpu.sync_copy(zbuf.at[pl.ds(0, 8)], acc.at[pl.ds(toff, 8)])

            plsc.subcore_barrier()

            @pl.loop(0, 4)
            def _(z):
                g_start(z, z)

            @pl.loop(0, NB_T)
            def _(i):
                k = lax.rem(i, 8)
                g_wait(i, k)
                scale(k, i)
                s_start(i, k)
                jn = i + 4
                kn = lax.rem(jn, 8)

                @pl.when(jn < NB_T)
                def _():
                    @pl.when(jn >= 8)
                    def _():
                        s_wait(jn - 8, kn)

                    g_start(jn, kn)

            @pl.loop(0, 8)
            def _(k):
                s_wait(NB_T - 8 + k, k)

            plsc.subcore_barrier()
            out_c = out_hbm.at[fc]
            pltpu.sync_copy(acc.at[pl.ds(base, ROWS_T)],
                            out_c.at[pl.ds(base, ROWS_T)])

            @pl.when(tid < 2)
            def _():
                toff = pl.multiple_of(TAIL0 + tid * 8, 8)
                pltpu.sync_copy(acc.at[pl.ds(toff, 8)],
                                out_c.at[pl.ds(toff, 8)])

    return prop


_DOT = functools.partial(lax.dot_general,
                         dimension_numbers=(((1,), (0,)), ((), ())),
                         precision=lax.Precision.HIGHEST,
                         preferred_element_type=jnp.float32)


def _k0(x, sdeg):
    def body(s_ref, x_ref, dv_ref, g0_ref):
        deg = s_ref[0] + 1.0
        dv = jnp.where(deg > 0, lax.rsqrt(deg), 0.0)
        dv_ref[...] = dv
        for c in range(2):
            g0_ref[c] = dv * x_ref[:, CW * c:CW * (c + 1)]

    return pl.pallas_call(
        body,
        grid=(N // BM,),
        in_specs=[
            pl.BlockSpec((1, BM, CW), lambda i: (0, i, 0)),
            pl.BlockSpec((BM, 128), lambda i: (i, 0)),
        ],
        out_specs=[
            pl.BlockSpec((BM, CW), lambda i: (i, 0)),
            pl.BlockSpec((2, BM, CW), lambda i: (0, i, 0)),
        ],
        out_shape=[
            jax.ShapeDtypeStruct((N, CW), jnp.float32),
            jax.ShapeDtypeStruct((2, N, CW), jnp.float32),
        ],
    )(sdeg, x)


def _make_mid(cin, cout, relu, scale_out):
    npad = cout * CW

    def body(s_ref, g_ref, dv_ref, w_ref, b_ref, o_ref):
        dv = dv_ref[...]
        t = jnp.concatenate(
            [dv * (s_ref[k] + g_ref[k]) for k in range(cin)], axis=1)
        u = _DOT(t, w_ref[...]) + b_ref[0]
        if relu:
            u = jnp.maximum(u, 0.0)
        for c in range(cout):
            oc = u[:, CW * c:CW * (c + 1)]
            o_ref[c] = dv * oc if scale_out else oc

    def run(s, g, dv, w2, b):
        return pl.pallas_call(
            body,
            grid=(N // BM,),
            in_specs=[
                pl.BlockSpec((cin, BM, CW), lambda i: (0, i, 0)),
                pl.BlockSpec((cin, BM, CW), lambda i: (0, i, 0)),
                pl.BlockSpec((BM, CW), lambda i: (i, 0)),
                pl.BlockSpec((cin * CW, npad), lambda i: (0, 0)),
                pl.BlockSpec((1, npad), lambda i: (0, 0)),
            ],
            out_specs=pl.BlockSpec((cout, BM, CW), lambda i: (0, i, 0)),
            out_shape=jax.ShapeDtypeStruct((cout, N, CW), jnp.float32),
        )(s, g, dv, w2, b)

    return run


def _k_last_mm(h6, dv, w2):
    def body(h_ref, dv_ref, w_ref, o_ref):
        t = jnp.concatenate([h_ref[k] for k in range(16)], axis=1)
        acc = _DOT(t, w_ref[...])
        dv = dv_ref[...]
        for c in range(2):
            o_ref[c] = dv * acc[:, CW * c:CW * (c + 1)]

    return pl.pallas_call(
        body,
        grid=(N // BM,),
        in_specs=[
            pl.BlockSpec((16, BM, CW), lambda i: (0, i, 0)),
            pl.BlockSpec((BM, CW), lambda i: (i, 0)),
            pl.BlockSpec((16 * CW, 128), lambda i: (0, 0)),
        ],
        out_specs=pl.BlockSpec((2, BM, CW), lambda i: (0, i, 0)),
        out_shape=jax.ShapeDtypeStruct((2, N, CW), jnp.float32),
    )(h6, dv, w2)


def _k_final(s6, mscaled, dv, b):
    def body(s_ref, m_ref, dv_ref, b_ref, o_ref):
        dv = dv_ref[...]
        for c in range(2):
            o_ref[:, CW * c:CW * (c + 1)] = (
                dv * (s_ref[c] + m_ref[c]) + b_ref[0, CW * c:CW * (c + 1)])

    return pl.pallas_call(
        body,
        grid=(N // BM,),
        in_specs=[
            pl.BlockSpec((2, BM, CW), lambda i: (0, i, 0)),
            pl.BlockSpec((2, BM, CW), lambda i: (0, i, 0)),
            pl.BlockSpec((BM, CW), lambda i: (i, 0)),
            pl.BlockSpec((1, 128), lambda i: (0, 0)),
        ],
        out_specs=pl.BlockSpec((BM, 128), lambda i: (i, 0)),
        out_shape=jax.ShapeDtypeStruct((N, 128), jnp.float32),
    )(s6, mscaled, dv, b)


def kernel(x, edge_index, edge_weight, Ws, bs):
    f32 = jnp.float32
    src = edge_index[0].astype(jnp.int32)
    dst = edge_index[1].astype(jnp.int32)
    ew = edge_weight.astype(f32)
    pad = E_PAD - E
    src_m = jnp.concatenate([src, jnp.zeros((pad,), jnp.int32)]).reshape(NB_ALL, EB)
    dst_m = jnp.concatenate([dst, jnp.zeros((pad,), jnp.int32)]).reshape(NB_ALL, EB)
    ew_m = jnp.concatenate([ew, jnp.zeros((pad,), f32)]).reshape(NB_ALL, EB)

    w2s, b2s = [], []
    for i, (din, dout) in enumerate(_DIMS):
        kp, np_ = _rup(din), _rup(dout)
        w2s.append(jnp.zeros((kp, np_), f32).at[:din, :dout].set(Ws[i]))
        b2s.append(jnp.zeros((1, np_), f32).at[0, :dout].set(bs[i]))

    prop2 = _make_prop(2)
    prop8 = _make_prop(8)
    prop16 = _make_prop(16)

    ones_g = jnp.ones((2, N, CW), f32)
    sdeg = prop2(ones_g, src_m, dst_m, ew_m)
    dv, g = _k0(x, sdeg)

    mid0 = _make_mid(2, 8, True, True)
    mid1 = _make_mid(8, 16, True, True)
    mid2 = _make_mid(16, 16, True, True)
    mid5 = _make_mid(16, 16, True, False)

    s = prop2(g, src_m, dst_m, ew_m)
    g = mid0(s, g, dv, w2s[0], b2s[0])
    s = prop8(g, src_m, dst_m, ew_m)
    g = mid1(s, g, dv, w2s[1], b2s[1])
    for i in (2, 3, 4):
        s = prop16(g, src_m, dst_m, ew_m)
        g = mid2(s, g, dv, w2s[i], b2s[i])
    s = prop16(g, src_m, dst_m, ew_m)
    h6 = mid5(s, g, dv, w2s[5], b2s[5])
    mscaled = _k_last_mm(h6, dv, w2s[6])
    s6 = prop2(mscaled, src_m, dst_m, ew_m)
    return _k_final(s6, mscaled, dv, b2s[6])
